# baseline (speedup 1.0000x reference)
import functools

import jax
import jax.numpy as jnp
from jax import lax
from jax.experimental import pallas as pl
from jax.experimental.pallas import tpu as pltpu
from jax.experimental.pallas import tpu_sc as plsc

N = 10000
NP = 10240
E = 50000
R = 11
H = 64
CH = 128
NCORE = 2
NSUB = 16
NCH_SUB = 26
EP = NSUB * NCH_SUB * CH
ROWS_SUB = NP // NSUB
NBUF = 4
NCH_SUB2 = NCH_SUB // NCORE
BLK = 256
NBLK = NP // BLK

SRC_OF = [1, 3, 0, 2, 3, 1, 0, 1, 3, 2, 0]
OWNER = [0, 0, 0, 0, 0, 0, 1, 1, 1, 1, 1]


def _get_mesh():
    return plsc.VectorSubcoreMesh(core_axis_name="c", subcore_axis_name="s",
                                  num_cores=NCORE, num_subcores=NSUB)


def _sc_degrees(s_idx, d_idx):
    @functools.partial(
        pl.kernel,
        out_type=(jax.ShapeDtypeStruct((R, NP, 16), jnp.float32),
                  jax.ShapeDtypeStruct((R, NP, 16), jnp.float32),
                  jax.ShapeDtypeStruct((R, NSUB, NCH_SUB, CH), jnp.int32),
                  jax.ShapeDtypeStruct((R, NSUB, NCH_SUB, CH), jnp.int32)),
        mesh=_get_mesh(),
        compiler_params=pltpu.CompilerParams(use_tc_tiling_on_sc=False),
        scratch_types=[
            pltpu.VMEM((NCH_SUB, CH), jnp.int32),
            pltpu.VMEM((NCH_SUB, CH), jnp.int32),
            pltpu.VMEM((CH, 16), jnp.float32),
            pltpu.VMEM((ROWS_SUB, 16), jnp.float32),
            pltpu.VMEM_SHARED((NP, 16), jnp.float32),
            pltpu.VMEM_SHARED((NP, 16), jnp.float32),
        ],
    )
    def k(s_hbm, d_hbm, cs_hbm, cd_hbm, so_hbm, do_hbm, sidx, didx, ones, zb,
          acc_s, acc_d):
        cid = lax.axis_index("c")
        sid = lax.axis_index("s")
        row0 = sid * ROWS_SUB

        @pl.loop(0, CH)
        def _(i):
            ones[i, :] = jnp.full((16,), 1.0, jnp.float32)

        @pl.loop(0, ROWS_SUB)
        def _(i):
            zb[i, :] = jnp.zeros((16,), jnp.float32)

        for r in range(R):
            @pl.when(cid == OWNER[r])
            def _():
                pltpu.sync_copy(zb, acc_s.at[pl.ds(row0, ROWS_SUB)])
                pltpu.sync_copy(zb, acc_d.at[pl.ds(row0, ROWS_SUB)])
                pltpu.sync_copy(s_hbm.at[r, sid], sidx)
                pltpu.sync_copy(d_hbm.at[r, sid], didx)
                plsc.subcore_barrier()

                @pl.loop(0, NCH_SUB)
                def _(j):
                    pltpu.sync_copy(ones, acc_s.at[sidx.at[j]], add=True)
                    pltpu.sync_copy(ones, acc_d.at[didx.at[j]], add=True)

                plsc.subcore_barrier()
                pltpu.sync_copy(acc_s.at[pl.ds(row0, ROWS_SUB)],
                                cs_hbm.at[r, pl.ds(row0, ROWS_SUB)])
                pltpu.sync_copy(acc_d.at[pl.ds(row0, ROWS_SUB)],
                                cd_hbm.at[r, pl.ds(row0, ROWS_SUB)])

                pltpu.sync_copy(didx, do_hbm.at[r, sid])

                @pl.loop(0, NCH_SUB)
                def _(j):
                    for c4 in range(CH // 16):
                        sl = pl.ds(c4 * 16, 16)
                        sidx[j, sl] = sidx[j, sl] + jnp.full(
                            (16,), r * NP, jnp.int32)

                pltpu.sync_copy(sidx, so_hbm.at[r, sid])

    return k(s_idx, d_idx)


def _sc_edge_pass(z2d, s_idx_off, d_idx):
    @functools.partial(
        pl.kernel,
        out_type=jax.ShapeDtypeStruct((NCORE, R, NP, H), jnp.float32),
        mesh=_get_mesh(),
        compiler_params=pltpu.CompilerParams(use_tc_tiling_on_sc=False),
        scratch_types=[
            pltpu.VMEM((NCH_SUB2, CH), jnp.int32),
            pltpu.VMEM((NCH_SUB2, CH), jnp.int32),
            pltpu.VMEM((NBUF, CH, H), jnp.float32),
            pltpu.VMEM((CH, H), jnp.float32),
            pltpu.VMEM_SHARED((NP, H), jnp.float32),
            [pltpu.SemaphoreType.DMA] * NBUF,
        ],
    )
    def k(z_hbm, s_hbm, d_hbm, agg_hbm, sidx, didx, rows, zb, acc, gsem):
        cid = lax.axis_index("c")
        sid = lax.axis_index("s")
        row0 = sid * ROWS_SUB

        @pl.loop(0, CH)
        def _(i):
            for c4 in range(H // 16):
                zb[i, pl.ds(c4 * 16, 16)] = jnp.zeros((16,), jnp.float32)

        for r in range(R):
            rr = lax.rem(r + 6 * cid, R)
            for kz in range(ROWS_SUB // CH):
                pltpu.sync_copy(zb, acc.at[pl.ds(row0 + kz * CH, CH)])
            pltpu.sync_copy(s_hbm.at[rr, cid, sid], sidx)
            pltpu.sync_copy(d_hbm.at[rr, cid, sid], didx)
            plsc.subcore_barrier()

            def wait_gather(c, b):
                pltpu.make_async_copy(z_hbm.at[sidx.at[c]], rows.at[b],
                                      gsem[b]).wait()

            for b in range(NBUF):
                pltpu.async_copy(z_hbm.at[sidx.at[b]], rows.at[b], gsem[b])

            @pl.loop(0, NCH_SUB2 - 1, step=NBUF)
            def _(jj):
                for b in range(NBUF):
                    c = jj + b
                    wait_gather(c, b)
                    pltpu.sync_copy(rows.at[b], acc.at[didx.at[c]],
                                    add=True)

                    @pl.when(c + NBUF < NCH_SUB2)
                    def _():
                        pltpu.async_copy(z_hbm.at[sidx.at[c + NBUF]],
                                         rows.at[b], gsem[b])

            c = NCH_SUB2 - 1
            wait_gather(c, c % NBUF)
            pltpu.sync_copy(rows.at[c % NBUF], acc.at[didx.at[c]], add=True)

            plsc.subcore_barrier()
            pltpu.sync_copy(acc.at[pl.ds(row0, ROWS_SUB)],
                            agg_hbm.at[cid, rr, pl.ds(row0, ROWS_SUB)])

    return k(z2d, s_idx_off, d_idx)


def _tc_z1(x0, x1, x2, x3, w1, cnt):
    def body(x0_ref, x1_ref, x2_ref, x3_ref, w_ref, c_ref, o_ref):
        xs = [x0_ref[...], x1_ref[...], x2_ref[...], x3_ref[...]]
        scl = lax.rsqrt(jnp.clip(c_ref[:, 0, :], 1.0, None))
        for r in range(R):
            y = jnp.dot(xs[SRC_OF[r]], w_ref[r],
                        preferred_element_type=jnp.float32)
            o_ref[r] = y * scl[r][:, None]

    x_bs = pl.BlockSpec((BLK, 128), lambda i: (i, 0))
    return pl.pallas_call(
        body,
        grid=(NBLK,),
        in_specs=[
            x_bs, x_bs, x_bs, x_bs,
            pl.BlockSpec((R, 128, H), lambda i: (0, 0, 0)),
            pl.BlockSpec((R, 1, BLK), lambda i: (0, 0, i)),
        ],
        out_specs=pl.BlockSpec((R, BLK, H), lambda i: (0, i, 0)),
        out_shape=jax.ShapeDtypeStruct((R, NP, H), jnp.float32),
    )(x0, x1, x2, x3, w1, cnt)


def _combine(agg, cnt_d, b):
    scl = lax.rsqrt(jnp.clip(cnt_d[:, 0, :], 1.0, None))
    conv = agg * scl[:, :, None] + b[:, None, :]
    rl = jax.nn.relu
    z = jnp.zeros((conv.shape[1], H), jnp.float32)
    h_cls = jnp.concatenate([rl(conv[0]), z, rl(conv[1])], axis=1)
    h_att = jnp.concatenate(
        [z, rl(jnp.maximum(jnp.maximum(conv[2], conv[3]), conv[4])), rl(conv[5])],
        axis=1)
    h_typ = jnp.concatenate(
        [z, rl(jnp.maximum(jnp.maximum(conv[6], conv[7]), conv[8])), rl(conv[9])],
        axis=1)
    h_val = jnp.concatenate([z, z, rl(conv[10])], axis=1)
    return [h_cls, h_att, h_typ, h_val]


def _tc_mid(agg1, cnt_d, cnt_s, b1, w2):
    def body(a_ref, cd_ref, cs_ref, b_ref, w_ref, o_ref):
        hs = _combine(a_ref[0] + a_ref[1], cd_ref[...], b_ref[...])
        scl_s = lax.rsqrt(jnp.clip(cs_ref[:, 0, :], 1.0, None))
        for r in range(R):
            y = jnp.dot(hs[SRC_OF[r]], w_ref[r],
                        preferred_element_type=jnp.float32)
            o_ref[r] = y * scl_s[r][:, None]

    return pl.pallas_call(
        body,
        grid=(NBLK,),
        in_specs=[
            pl.BlockSpec((NCORE, R, BLK, H), lambda i: (0, 0, i, 0)),
            pl.BlockSpec((R, 1, BLK), lambda i: (0, 0, i)),
            pl.BlockSpec((R, 1, BLK), lambda i: (0, 0, i)),
            pl.BlockSpec((R, H), lambda i: (0, 0)),
            pl.BlockSpec((R, 3 * H, H), lambda i: (0, 0, 0)),
        ],
        out_specs=pl.BlockSpec((R, BLK, H), lambda i: (0, i, 0)),
        out_shape=jax.ShapeDtypeStruct((R, NP, H), jnp.float32),
    )(agg1, cnt_d, cnt_s, b1, w2)


def _tc_final(agg2, cnt_d, b2):
    def body(a_ref, cd_ref, b_ref, oc_ref, oa_ref, ot_ref, ov_ref):
        hs = _combine(a_ref[0] + a_ref[1], cd_ref[...], b_ref[...])
        oc_ref[...], oa_ref[...], ot_ref[...], ov_ref[...] = hs

    out_bs = pl.BlockSpec((BLK, 3 * H), lambda i: (i, 0))
    out_t = jax.ShapeDtypeStruct((NP, 3 * H), jnp.float32)
    return pl.pallas_call(
        body,
        grid=(NBLK,),
        in_specs=[
            pl.BlockSpec((NCORE, R, BLK, H), lambda i: (0, 0, i, 0)),
            pl.BlockSpec((R, 1, BLK), lambda i: (0, 0, i)),
            pl.BlockSpec((R, H), lambda i: (0, 0)),
        ],
        out_specs=(out_bs, out_bs, out_bs, out_bs),
        out_shape=(out_t, out_t, out_t, out_t),
    )(agg2, cnt_d, b2)


def kernel(x_class, x_attribute, x_type, x_value,
           edges_a2c, edges_v2c, edges_c2a, edges_t2a, edges_v2a, edges_a2a,
           edges_c2t, edges_a2t, edges_v2t, edges_t2t, edges_c2v,
           W1, b1, W2, b2):
    edges = [edges_a2c, edges_v2c, edges_c2a, edges_t2a, edges_v2a, edges_a2a,
             edges_c2t, edges_a2t, edges_v2t, edges_t2t, edges_c2v]

    pad = jnp.full((2, EP - E), N, dtype=jnp.int32)
    ee = jnp.stack([jnp.concatenate([e.astype(jnp.int32), pad], axis=1)
                    for e in edges])
    s_idx = ee[:, 0, :].reshape(R, NSUB, NCH_SUB, CH)
    d_idx = ee[:, 1, :].reshape(R, NSUB, NCH_SUB, CH)

    xs = [x_class, x_attribute, x_type, x_value]
    xs = [jnp.pad(x[:N], ((0, NP - N), (0, 0))) for x in xs]

    cs, cd, s_idx_off, d_idx = _sc_degrees(s_idx, d_idx)
    s_idx_off = s_idx_off.reshape(R, NCORE, NSUB, NCH_SUB2, CH)
    d_idx = d_idx.reshape(R, NCORE, NSUB, NCH_SUB2, CH)
    cnt_s = cs[:, :, 0].reshape(R, 1, NP)
    cnt_d = cd[:, :, 0].reshape(R, 1, NP)

    z1 = _tc_z1(xs[0], xs[1], xs[2], xs[3], W1, cnt_s).reshape(R * NP, H)

    agg1 = _sc_edge_pass(z1, s_idx_off, d_idx)

    z2 = _tc_mid(agg1, cnt_d, cnt_s, b1, W2).reshape(R * NP, H)

    agg2 = _sc_edge_pass(z2, s_idx_off, d_idx)

    oc, oa, ot, ov = _tc_final(agg2, cnt_d, b2)

    za = jnp.zeros((H,), jnp.float32)
    att_const = jax.nn.relu(jnp.concatenate(
        [za, jnp.maximum(jnp.maximum(b2[2], b2[3]), b2[4]), b2[5]]))
    attr_tail = jnp.broadcast_to(att_const, (N, 3 * H))

    return (oc[:N], jnp.concatenate([oa[:N], attr_tail], axis=0),
            ot[:N], ov[:N])


# speedup vs baseline: 1.5429x; 1.5429x over previous
import functools

import jax
import jax.numpy as jnp
from jax import lax
from jax.experimental import pallas as pl
from jax.experimental.pallas import tpu as pltpu
from jax.experimental.pallas import tpu_sc as plsc

N = 10000
NP = 10240
E = 50000
R = 11
H = 64
CH = 128
NCORE = 2
NSUB = 16
NCH_SUB = 26
EP = NSUB * NCH_SUB * CH
ROWS_SUB = NP // NSUB
NBUF = 4
BLK = 256
NBLK = NP // BLK

SRC_OF = [1, 3, 0, 2, 3, 1, 0, 1, 3, 2, 0]
OWNER = [0, 0, 0, 0, 0, 0, 1, 1, 1, 1, 1]


def _get_mesh():
    return plsc.VectorSubcoreMesh(core_axis_name="c", subcore_axis_name="s",
                                  num_cores=NCORE, num_subcores=NSUB)


def _sc_degrees(s_idx, d_idx):
    @functools.partial(
        pl.kernel,
        out_type=(jax.ShapeDtypeStruct((R, NP, 16), jnp.float32),
                  jax.ShapeDtypeStruct((R, NP, 16), jnp.float32),
                  jax.ShapeDtypeStruct((R, NSUB, NCH_SUB, CH), jnp.int32),
                  jax.ShapeDtypeStruct((R, NSUB, NCH_SUB, CH), jnp.int32)),
        mesh=_get_mesh(),
        compiler_params=pltpu.CompilerParams(use_tc_tiling_on_sc=False),
        scratch_types=[
            pltpu.VMEM((NCH_SUB, CH), jnp.int32),
            pltpu.VMEM((NCH_SUB, CH), jnp.int32),
            pltpu.VMEM((CH, 16), jnp.float32),
            pltpu.VMEM((ROWS_SUB, 16), jnp.float32),
            pltpu.VMEM_SHARED((NP, 16), jnp.float32),
            pltpu.VMEM_SHARED((NP, 16), jnp.float32),
            pltpu.SemaphoreType.DMA,
            pltpu.SemaphoreType.DMA,
        ],
    )
    def k(s_hbm, d_hbm, cs_hbm, cd_hbm, so_hbm, do_hbm, sidx, didx, ones, zb,
          acc_s, acc_d, sem_s, sem_d):
        cid = lax.axis_index("c")
        sid = lax.axis_index("s")
        row0 = sid * ROWS_SUB

        @pl.loop(0, CH)
        def _(i):
            ones[i, :] = jnp.full((16,), 1.0, jnp.float32)

        @pl.loop(0, ROWS_SUB)
        def _(i):
            zb[i, :] = jnp.zeros((16,), jnp.float32)

        for r in range(R):
            @pl.when(cid == OWNER[r])
            def _():
                pltpu.sync_copy(zb, acc_s.at[pl.ds(row0, ROWS_SUB)])
                pltpu.sync_copy(zb, acc_d.at[pl.ds(row0, ROWS_SUB)])
                pltpu.sync_copy(s_hbm.at[r, sid], sidx)
                pltpu.sync_copy(d_hbm.at[r, sid], didx)
                plsc.subcore_barrier()

                @pl.loop(0, NCH_SUB)
                def _(j):
                    pltpu.async_copy(ones, acc_s.at[sidx.at[j]], sem_s,
                                     add=True)
                    pltpu.async_copy(ones, acc_d.at[didx.at[j]], sem_d,
                                     add=True)
                    pltpu.make_async_copy(ones, acc_s.at[sidx.at[j]],
                                          sem_s).wait()
                    pltpu.make_async_copy(ones, acc_d.at[didx.at[j]],
                                          sem_d).wait()

                plsc.subcore_barrier()
                pltpu.sync_copy(acc_s.at[pl.ds(row0, ROWS_SUB)],
                                cs_hbm.at[r, pl.ds(row0, ROWS_SUB)])
                pltpu.sync_copy(acc_d.at[pl.ds(row0, ROWS_SUB)],
                                cd_hbm.at[r, pl.ds(row0, ROWS_SUB)])

                pltpu.sync_copy(didx, do_hbm.at[r, sid])

                @pl.loop(0, NCH_SUB)
                def _(j):
                    for c4 in range(CH // 16):
                        sl = pl.ds(c4 * 16, 16)
                        sidx[j, sl] = sidx[j, sl] + jnp.full(
                            (16,), r * NP, jnp.int32)

                pltpu.sync_copy(sidx, so_hbm.at[r, sid])

    return k(s_idx, d_idx)


def _sc_edge_pass(z2d, s_idx_off, d_idx):
    @functools.partial(
        pl.kernel,
        out_type=jax.ShapeDtypeStruct((R, NP, H), jnp.float32),
        mesh=_get_mesh(),
        compiler_params=pltpu.CompilerParams(use_tc_tiling_on_sc=False),
        scratch_types=[
            pltpu.VMEM((NCH_SUB, CH), jnp.int32),
            pltpu.VMEM((NCH_SUB, CH), jnp.int32),
            pltpu.VMEM((NBUF, CH, H), jnp.float32),
            pltpu.VMEM((CH, H), jnp.float32),
            pltpu.VMEM_SHARED((NP, H), jnp.float32),
            [pltpu.SemaphoreType.DMA] * NBUF,
        ],
    )
    def k(z_hbm, s_hbm, d_hbm, agg_hbm, sidx, didx, rows, zb, acc, gsem):
        cid = lax.axis_index("c")
        sid = lax.axis_index("s")
        row0 = sid * ROWS_SUB

        @pl.loop(0, CH)
        def _(i):
            for c4 in range(H // 16):
                zb[i, pl.ds(c4 * 16, 16)] = jnp.zeros((16,), jnp.float32)

        for r in range(R):
            @pl.when(cid == OWNER[r])
            def _():
                for kz in range(ROWS_SUB // CH):
                    pltpu.sync_copy(zb, acc.at[pl.ds(row0 + kz * CH, CH)])
                pltpu.sync_copy(s_hbm.at[r, sid], sidx)
                pltpu.sync_copy(d_hbm.at[r, sid], didx)
                plsc.subcore_barrier()

                def wait_gather(c, b):
                    pltpu.make_async_copy(z_hbm.at[sidx.at[c]], rows.at[b],
                                          gsem[b]).wait()

                for b in range(NBUF):
                    pltpu.async_copy(z_hbm.at[sidx.at[b]], rows.at[b], gsem[b])

                @pl.loop(0, NCH_SUB - 2, step=NBUF)
                def _(jj):
                    for b in range(NBUF):
                        c = jj + b
                        wait_gather(c, b)
                        pltpu.sync_copy(rows.at[b], acc.at[didx.at[c]],
                                        add=True)

                        @pl.when(c + NBUF < NCH_SUB)
                        def _():
                            pltpu.async_copy(z_hbm.at[sidx.at[c + NBUF]],
                                             rows.at[b], gsem[b])

                for c in (NCH_SUB - 2, NCH_SUB - 1):
                    b = c % NBUF
                    wait_gather(c, b)
                    pltpu.sync_copy(rows.at[b], acc.at[didx.at[c]], add=True)

                plsc.subcore_barrier()
                pltpu.sync_copy(acc.at[pl.ds(row0, ROWS_SUB)],
                                agg_hbm.at[r, pl.ds(row0, ROWS_SUB)])

    return k(z2d, s_idx_off, d_idx)


def _tc_z1(x0, x1, x2, x3, w1, cnt):
    def body(x0_ref, x1_ref, x2_ref, x3_ref, w_ref, c_ref, o_ref):
        xs = [x0_ref[...], x1_ref[...], x2_ref[...], x3_ref[...]]
        scl = lax.rsqrt(jnp.clip(c_ref[:, 0, :], 1.0, None))
        for r in range(R):
            y = jnp.dot(xs[SRC_OF[r]], w_ref[r],
                        preferred_element_type=jnp.float32)
            o_ref[r] = y * scl[r][:, None]

    x_bs = pl.BlockSpec((BLK, 128), lambda i: (i, 0))
    return pl.pallas_call(
        body,
        grid=(NBLK,),
        in_specs=[
            x_bs, x_bs, x_bs, x_bs,
            pl.BlockSpec((R, 128, H), lambda i: (0, 0, 0)),
            pl.BlockSpec((R, 1, BLK), lambda i: (0, 0, i)),
        ],
        out_specs=pl.BlockSpec((R, BLK, H), lambda i: (0, i, 0)),
        out_shape=jax.ShapeDtypeStruct((R, NP, H), jnp.float32),
    )(x0, x1, x2, x3, w1, cnt)


def _combine(agg, cnt_d, b):
    scl = lax.rsqrt(jnp.clip(cnt_d[:, 0, :], 1.0, None))
    conv = agg * scl[:, :, None] + b[:, None, :]
    rl = jax.nn.relu
    z = jnp.zeros((conv.shape[1], H), jnp.float32)
    h_cls = jnp.concatenate([rl(conv[0]), z, rl(conv[1])], axis=1)
    h_att = jnp.concatenate(
        [z, rl(jnp.maximum(jnp.maximum(conv[2], conv[3]), conv[4])), rl(conv[5])],
        axis=1)
    h_typ = jnp.concatenate(
        [z, rl(jnp.maximum(jnp.maximum(conv[6], conv[7]), conv[8])), rl(conv[9])],
        axis=1)
    h_val = jnp.concatenate([z, z, rl(conv[10])], axis=1)
    return [h_cls, h_att, h_typ, h_val]


def _tc_mid(agg1, cnt_d, cnt_s, b1, w2):
    def body(a_ref, cd_ref, cs_ref, b_ref, w_ref, o_ref):
        hs = _combine(a_ref[...], cd_ref[...], b_ref[...])
        scl_s = lax.rsqrt(jnp.clip(cs_ref[:, 0, :], 1.0, None))
        for r in range(R):
            y = jnp.dot(hs[SRC_OF[r]], w_ref[r],
                        preferred_element_type=jnp.float32)
            o_ref[r] = y * scl_s[r][:, None]

    return pl.pallas_call(
        body,
        grid=(NBLK,),
        in_specs=[
            pl.BlockSpec((R, BLK, H), lambda i: (0, i, 0)),
            pl.BlockSpec((R, 1, BLK), lambda i: (0, 0, i)),
            pl.BlockSpec((R, 1, BLK), lambda i: (0, 0, i)),
            pl.BlockSpec((R, H), lambda i: (0, 0)),
            pl.BlockSpec((R, 3 * H, H), lambda i: (0, 0, 0)),
        ],
        out_specs=pl.BlockSpec((R, BLK, H), lambda i: (0, i, 0)),
        out_shape=jax.ShapeDtypeStruct((R, NP, H), jnp.float32),
    )(agg1, cnt_d, cnt_s, b1, w2)


def _tc_final(agg2, cnt_d, b2):
    def body(a_ref, cd_ref, b_ref, oc_ref, oa_ref, ot_ref, ov_ref):
        hs = _combine(a_ref[...], cd_ref[...], b_ref[...])
        oc_ref[...], oa_ref[...], ot_ref[...], ov_ref[...] = hs

    out_bs = pl.BlockSpec((BLK, 3 * H), lambda i: (i, 0))
    out_t = jax.ShapeDtypeStruct((NP, 3 * H), jnp.float32)
    return pl.pallas_call(
        body,
        grid=(NBLK,),
        in_specs=[
            pl.BlockSpec((R, BLK, H), lambda i: (0, i, 0)),
            pl.BlockSpec((R, 1, BLK), lambda i: (0, 0, i)),
            pl.BlockSpec((R, H), lambda i: (0, 0)),
        ],
        out_specs=(out_bs, out_bs, out_bs, out_bs),
        out_shape=(out_t, out_t, out_t, out_t),
    )(agg2, cnt_d, b2)


def kernel(x_class, x_attribute, x_type, x_value,
           edges_a2c, edges_v2c, edges_c2a, edges_t2a, edges_v2a, edges_a2a,
           edges_c2t, edges_a2t, edges_v2t, edges_t2t, edges_c2v,
           W1, b1, W2, b2):
    edges = [edges_a2c, edges_v2c, edges_c2a, edges_t2a, edges_v2a, edges_a2a,
             edges_c2t, edges_a2t, edges_v2t, edges_t2t, edges_c2v]

    pad = jnp.full((2, EP - E), N, dtype=jnp.int32)
    ee = jnp.stack([jnp.concatenate([e.astype(jnp.int32), pad], axis=1)
                    for e in edges])
    s_idx = ee[:, 0, :].reshape(R, NSUB, NCH_SUB, CH)
    d_idx = ee[:, 1, :].reshape(R, NSUB, NCH_SUB, CH)

    xs = [x_class, x_attribute, x_type, x_value]
    xs = [jnp.pad(x[:N], ((0, NP - N), (0, 0))) for x in xs]

    cs, cd, s_idx_off, d_idx = _sc_degrees(s_idx, d_idx)
    cnt_s = cs[:, :, 0].reshape(R, 1, NP)
    cnt_d = cd[:, :, 0].reshape(R, 1, NP)

    z1 = _tc_z1(xs[0], xs[1], xs[2], xs[3], W1, cnt_s).reshape(R * NP, H)

    agg1 = _sc_edge_pass(z1, s_idx_off, d_idx)

    z2 = _tc_mid(agg1, cnt_d, cnt_s, b1, W2).reshape(R * NP, H)

    agg2 = _sc_edge_pass(z2, s_idx_off, d_idx)

    oc, oa, ot, ov = _tc_final(agg2, cnt_d, b2)

    za = jnp.zeros((H,), jnp.float32)
    att_const = jax.nn.relu(jnp.concatenate(
        [za, jnp.maximum(jnp.maximum(b2[2], b2[3]), b2[4]), b2[5]]))
    attr_tail = jnp.broadcast_to(att_const, (N, 3 * H))

    return (oc[:N], jnp.concatenate([oa[:N], attr_tail], axis=0),
            ot[:N], ov[:N])


# speedup vs baseline: 2.2063x; 1.4299x over previous
import functools

import jax
import jax.numpy as jnp
from jax import lax
from jax.experimental import pallas as pl
from jax.experimental.pallas import tpu as pltpu
from jax.experimental.pallas import tpu_sc as plsc

N = 10000
NP = 10240
E = 50000
R = 11
H = 64
CH = 128
NCORE = 2
NSUB = 16
NCH_SUB = 25
EP = NSUB * NCH_SUB * CH
ROWS_SUB = NP // NSUB
NBUF = 4
BLK = 256
NBLK = NP // BLK

SRC_OF = [1, 3, 0, 2, 3, 1, 0, 1, 3, 2, 0]
OWNER = [0, 0, 0, 0, 0, 0, 1, 1, 1, 1, 1]


def _get_mesh():
    return plsc.VectorSubcoreMesh(core_axis_name="c", subcore_axis_name="s",
                                  num_cores=NCORE, num_subcores=NSUB)


def _sc_degrees(s_idx, d_idx):
    @functools.partial(
        pl.kernel,
        out_type=(jax.ShapeDtypeStruct((R, NP, 16), jnp.float32),
                  jax.ShapeDtypeStruct((R, NP, 16), jnp.float32),
                  jax.ShapeDtypeStruct((R, NSUB, NCH_SUB, CH), jnp.int32),
                  jax.ShapeDtypeStruct((R, NSUB, NCH_SUB, CH), jnp.int32)),
        mesh=_get_mesh(),
        compiler_params=pltpu.CompilerParams(use_tc_tiling_on_sc=False),
        scratch_types=[
            pltpu.VMEM((NCH_SUB, CH), jnp.int32),
            pltpu.VMEM((NCH_SUB, CH), jnp.int32),
            pltpu.VMEM((CH, 16), jnp.float32),
            pltpu.VMEM((ROWS_SUB, 16), jnp.float32),
            pltpu.VMEM_SHARED((NP, 16), jnp.float32),
            pltpu.VMEM_SHARED((NP, 16), jnp.float32),
            pltpu.SemaphoreType.DMA,
            pltpu.SemaphoreType.DMA,
        ],
    )
    def k(s_hbm, d_hbm, cs_hbm, cd_hbm, so_hbm, do_hbm, sidx, didx, ones, zb,
          acc_s, acc_d, sem_s, sem_d):
        cid = lax.axis_index("c")
        sid = lax.axis_index("s")
        row0 = sid * ROWS_SUB

        @pl.loop(0, CH)
        def _(i):
            ones[i, :] = jnp.full((16,), 1.0, jnp.float32)

        @pl.loop(0, ROWS_SUB)
        def _(i):
            zb[i, :] = jnp.zeros((16,), jnp.float32)

        for r in range(R):
            @pl.when(cid == OWNER[r])
            def _():
                pltpu.sync_copy(zb, acc_s.at[pl.ds(row0, ROWS_SUB)])
                pltpu.sync_copy(zb, acc_d.at[pl.ds(row0, ROWS_SUB)])
                pltpu.sync_copy(s_hbm.at[r, sid], sidx)
                pltpu.sync_copy(d_hbm.at[r, sid], didx)
                plsc.subcore_barrier()

                @pl.loop(0, NCH_SUB)
                def _(j):
                    pltpu.async_copy(ones, acc_s.at[sidx.at[j]], sem_s,
                                     add=True)
                    pltpu.async_copy(ones, acc_d.at[didx.at[j]], sem_d,
                                     add=True)
                    pltpu.make_async_copy(ones, acc_s.at[sidx.at[j]],
                                          sem_s).wait()
                    pltpu.make_async_copy(ones, acc_d.at[didx.at[j]],
                                          sem_d).wait()

                plsc.subcore_barrier()
                pltpu.sync_copy(acc_s.at[pl.ds(row0, ROWS_SUB)],
                                cs_hbm.at[r, pl.ds(row0, ROWS_SUB)])
                pltpu.sync_copy(acc_d.at[pl.ds(row0, ROWS_SUB)],
                                cd_hbm.at[r, pl.ds(row0, ROWS_SUB)])

                pltpu.sync_copy(didx, do_hbm.at[r, sid])

                @pl.loop(0, NCH_SUB)
                def _(j):
                    for c4 in range(CH // 16):
                        sl = pl.ds(c4 * 16, 16)
                        sidx[j, sl] = sidx[j, sl] + jnp.full(
                            (16,), r * NP, jnp.int32)

                pltpu.sync_copy(sidx, so_hbm.at[r, sid])

    return k(s_idx, d_idx)


def _sc_edge_pass(z2d, s_idx_off, d_idx):
    @functools.partial(
        pl.kernel,
        out_type=jax.ShapeDtypeStruct((R, NP, H), jnp.float32),
        mesh=_get_mesh(),
        compiler_params=pltpu.CompilerParams(use_tc_tiling_on_sc=False),
        scratch_types=[
            pltpu.VMEM((NCH_SUB, CH), jnp.int32),
            pltpu.VMEM((NCH_SUB, CH), jnp.int32),
            pltpu.VMEM((NBUF, CH, H), jnp.float32),
            pltpu.VMEM((CH, H), jnp.float32),
            pltpu.VMEM_SHARED((NP, H), jnp.float32),
            [pltpu.SemaphoreType.DMA] * NBUF,
        ],
    )
    def k(z_hbm, s_hbm, d_hbm, agg_hbm, sidx, didx, rows, zb, acc, gsem):
        cid = lax.axis_index("c")
        sid = lax.axis_index("s")
        row0 = sid * ROWS_SUB

        @pl.loop(0, CH)
        def _(i):
            for c4 in range(H // 16):
                zb[i, pl.ds(c4 * 16, 16)] = jnp.zeros((16,), jnp.float32)

        for r in range(R):
            @pl.when(cid == OWNER[r])
            def _():
                for kz in range(ROWS_SUB // CH):
                    pltpu.sync_copy(zb, acc.at[pl.ds(row0 + kz * CH, CH)])
                pltpu.sync_copy(s_hbm.at[r, sid], sidx)
                pltpu.sync_copy(d_hbm.at[r, sid], didx)
                plsc.subcore_barrier()

                def wait_gather(c, b):
                    pltpu.make_async_copy(z_hbm.at[sidx.at[c]], rows.at[b],
                                          gsem[b]).wait()

                for b in range(NBUF):
                    pltpu.async_copy(z_hbm.at[sidx.at[b]], rows.at[b], gsem[b])

                @pl.loop(0, (NCH_SUB // NBUF) * NBUF, step=NBUF)
                def _(jj):
                    for b in range(NBUF):
                        c = jj + b
                        wait_gather(c, b)
                        pltpu.sync_copy(rows.at[b], acc.at[didx.at[c]],
                                        add=True)

                        @pl.when(c + NBUF < NCH_SUB)
                        def _():
                            pltpu.async_copy(z_hbm.at[sidx.at[c + NBUF]],
                                             rows.at[b], gsem[b])

                for c in range((NCH_SUB // NBUF) * NBUF, NCH_SUB):
                    b = c % NBUF
                    wait_gather(c, b)
                    pltpu.sync_copy(rows.at[b], acc.at[didx.at[c]], add=True)

                plsc.subcore_barrier()
                pltpu.sync_copy(acc.at[pl.ds(row0, ROWS_SUB)],
                                agg_hbm.at[r, pl.ds(row0, ROWS_SUB)])

    return k(z2d, s_idx_off, d_idx)


def _tc_z1(x0, x1, x2, x3, w1, cnt):
    def body(x0_ref, x1_ref, x2_ref, x3_ref, w_ref, c_ref, o_ref):
        xs = [x0_ref[...], x1_ref[...], x2_ref[...], x3_ref[...]]
        scl = lax.rsqrt(jnp.clip(c_ref[:, 0, :], 1.0, None))
        for r in range(R):
            y = jnp.dot(xs[SRC_OF[r]], w_ref[r],
                        preferred_element_type=jnp.float32)
            o_ref[r] = y * scl[r][:, None]

    x_bs = pl.BlockSpec((BLK, 128), lambda i: (i, 0))
    return pl.pallas_call(
        body,
        grid=(NBLK,),
        in_specs=[
            x_bs, x_bs, x_bs, x_bs,
            pl.BlockSpec((R, 128, H), lambda i: (0, 0, 0)),
            pl.BlockSpec((R, 1, BLK), lambda i: (0, 0, i)),
        ],
        out_specs=pl.BlockSpec((R, BLK, H), lambda i: (0, i, 0)),
        out_shape=jax.ShapeDtypeStruct((R, NP, H), jnp.float32),
    )(x0, x1, x2, x3, w1, cnt)


def _combine(agg, cnt_d, b):
    scl = lax.rsqrt(jnp.clip(cnt_d[:, 0, :], 1.0, None))
    conv = agg * scl[:, :, None] + b[:, None, :]
    rl = jax.nn.relu
    z = jnp.zeros((conv.shape[1], H), jnp.float32)
    h_cls = jnp.concatenate([rl(conv[0]), z, rl(conv[1])], axis=1)
    h_att = jnp.concatenate(
        [z, rl(jnp.maximum(jnp.maximum(conv[2], conv[3]), conv[4])), rl(conv[5])],
        axis=1)
    h_typ = jnp.concatenate(
        [z, rl(jnp.maximum(jnp.maximum(conv[6], conv[7]), conv[8])), rl(conv[9])],
        axis=1)
    h_val = jnp.concatenate([z, z, rl(conv[10])], axis=1)
    return [h_cls, h_att, h_typ, h_val]


def _tc_mid(agg1, cnt_d, cnt_s, b1, w2):
    def body(a_ref, cd_ref, cs_ref, b_ref, w_ref, o_ref):
        hs = _combine(a_ref[...], cd_ref[...], b_ref[...])
        scl_s = lax.rsqrt(jnp.clip(cs_ref[:, 0, :], 1.0, None))
        for r in range(R):
            y = jnp.dot(hs[SRC_OF[r]], w_ref[r],
                        preferred_element_type=jnp.float32)
            o_ref[r] = y * scl_s[r][:, None]

    return pl.pallas_call(
        body,
        grid=(NBLK,),
        in_specs=[
            pl.BlockSpec((R, BLK, H), lambda i: (0, i, 0)),
            pl.BlockSpec((R, 1, BLK), lambda i: (0, 0, i)),
            pl.BlockSpec((R, 1, BLK), lambda i: (0, 0, i)),
            pl.BlockSpec((R, H), lambda i: (0, 0)),
            pl.BlockSpec((R, 3 * H, H), lambda i: (0, 0, 0)),
        ],
        out_specs=pl.BlockSpec((R, BLK, H), lambda i: (0, i, 0)),
        out_shape=jax.ShapeDtypeStruct((R, NP, H), jnp.float32),
    )(agg1, cnt_d, cnt_s, b1, w2)


def _tc_final(agg2, cnt_d, b2):
    def body(a_ref, cd_ref, b_ref, oc_ref, oa_ref, ot_ref, ov_ref):
        hs = _combine(a_ref[...], cd_ref[...], b_ref[...])
        oc_ref[...], oa_ref[...], ot_ref[...], ov_ref[...] = hs

    out_bs = pl.BlockSpec((BLK, 3 * H), lambda i: (i, 0))
    out_t = jax.ShapeDtypeStruct((NP, 3 * H), jnp.float32)
    return pl.pallas_call(
        body,
        grid=(NBLK,),
        in_specs=[
            pl.BlockSpec((R, BLK, H), lambda i: (0, i, 0)),
            pl.BlockSpec((R, 1, BLK), lambda i: (0, 0, i)),
            pl.BlockSpec((R, H), lambda i: (0, 0)),
        ],
        out_specs=(out_bs, out_bs, out_bs, out_bs),
        out_shape=(out_t, out_t, out_t, out_t),
    )(agg2, cnt_d, b2)


def kernel(x_class, x_attribute, x_type, x_value,
           edges_a2c, edges_v2c, edges_c2a, edges_t2a, edges_v2a, edges_a2a,
           edges_c2t, edges_a2t, edges_v2t, edges_t2t, edges_c2v,
           W1, b1, W2, b2):
    edges = [edges_a2c, edges_v2c, edges_c2a, edges_t2a, edges_v2a, edges_a2a,
             edges_c2t, edges_a2t, edges_v2t, edges_t2t, edges_c2v]

    pad = jnp.full((2, EP - E), N, dtype=jnp.int32)
    ee = jnp.stack([jnp.concatenate([e.astype(jnp.int32), pad], axis=1)
                    for e in edges])
    s_idx = ee[:, 0, :].reshape(R, NSUB, NCH_SUB, CH)
    d_idx = ee[:, 1, :].reshape(R, NSUB, NCH_SUB, CH)

    xs = [x_class, x_attribute, x_type, x_value]
    xs = [jnp.pad(x[:N], ((0, NP - N), (0, 0))) for x in xs]

    cs, cd, s_idx_off, d_idx = _sc_degrees(s_idx, d_idx)
    cnt_s = cs[:, :, 0].reshape(R, 1, NP)
    cnt_d = cd[:, :, 0].reshape(R, 1, NP)

    z1 = _tc_z1(xs[0], xs[1], xs[2], xs[3], W1, cnt_s).reshape(R * NP, H)

    agg1 = _sc_edge_pass(z1, s_idx_off, d_idx)

    z2 = _tc_mid(agg1, cnt_d, cnt_s, b1, W2).reshape(R * NP, H)

    agg2 = _sc_edge_pass(z2, s_idx_off, d_idx)

    oc, oa, ot, ov = _tc_final(agg2, cnt_d, b2)

    za = jnp.zeros((H,), jnp.float32)
    att_const = jax.nn.relu(jnp.concatenate(
        [za, jnp.maximum(jnp.maximum(b2[2], b2[3]), b2[4]), b2[5]]))
    attr_tail = jnp.broadcast_to(att_const, (N, 3 * H))

    return (oc[:N], jnp.concatenate([oa[:N], attr_tail], axis=0),
            ot[:N], ov[:N])


# speedup vs baseline: 2.9437x; 1.3342x over previous
import functools

import jax
import jax.numpy as jnp
from jax import lax
from jax.experimental import pallas as pl
from jax.experimental.pallas import tpu as pltpu
from jax.experimental.pallas import tpu_sc as plsc

N = 10000
NP = 10240
E = 50000
R = 11
H = 64
CH = 128
NCORE = 2
NSUB = 16
NCH_SUB = 25
EP = NSUB * NCH_SUB * CH
ROWS_SUB = NP // NSUB
NBUF = 4
BLK = 256
NBLK = NP // BLK

SRC_OF = [1, 3, 0, 2, 3, 1, 0, 1, 3, 2, 0]
OWNER = [0, 0, 0, 0, 0, 0, 1, 1, 1, 1, 1]


def _get_mesh():
    return plsc.VectorSubcoreMesh(core_axis_name="c", subcore_axis_name="s",
                                  num_cores=NCORE, num_subcores=NSUB)


def _sc_degrees(s_idx, d_idx):
    @functools.partial(
        pl.kernel,
        out_type=(jax.ShapeDtypeStruct((R, NP, 16), jnp.float32),
                  jax.ShapeDtypeStruct((R, NP, 16), jnp.float32),
                  jax.ShapeDtypeStruct((R, NSUB, NCH_SUB, CH), jnp.int32),
                  jax.ShapeDtypeStruct((R, NSUB, NCH_SUB, CH), jnp.int32)),
        mesh=_get_mesh(),
        compiler_params=pltpu.CompilerParams(use_tc_tiling_on_sc=False),
        scratch_types=[
            pltpu.VMEM((NCH_SUB, CH), jnp.int32),
            pltpu.VMEM((NCH_SUB, CH), jnp.int32),
            pltpu.VMEM((CH, 16), jnp.float32),
            pltpu.VMEM((ROWS_SUB, 16), jnp.float32),
            pltpu.VMEM_SHARED((NP, 16), jnp.float32),
            pltpu.VMEM_SHARED((NP, 16), jnp.float32),
            pltpu.SemaphoreType.DMA,
            pltpu.SemaphoreType.DMA,
        ],
    )
    def k(s_hbm, d_hbm, cs_hbm, cd_hbm, so_hbm, do_hbm, sidx, didx, ones, zb,
          acc_s, acc_d, sem_s, sem_d):
        cid = lax.axis_index("c")
        sid = lax.axis_index("s")
        row0 = sid * ROWS_SUB

        @pl.loop(0, CH)
        def _(i):
            ones[i, :] = jnp.full((16,), 1.0, jnp.float32)

        @pl.loop(0, ROWS_SUB)
        def _(i):
            zb[i, :] = jnp.zeros((16,), jnp.float32)

        for r in range(R):
            @pl.when(cid == OWNER[r])
            def _():
                pltpu.sync_copy(zb, acc_s.at[pl.ds(row0, ROWS_SUB)])
                pltpu.sync_copy(zb, acc_d.at[pl.ds(row0, ROWS_SUB)])
                pltpu.sync_copy(s_hbm.at[r, sid], sidx)
                pltpu.sync_copy(d_hbm.at[r, sid], didx)
                plsc.subcore_barrier()

                @pl.loop(0, NCH_SUB)
                def _(j):
                    pltpu.async_copy(ones, acc_s.at[sidx.at[j]], sem_s,
                                     add=True)
                    pltpu.async_copy(ones, acc_d.at[didx.at[j]], sem_d,
                                     add=True)
                    pltpu.make_async_copy(ones, acc_s.at[sidx.at[j]],
                                          sem_s).wait()
                    pltpu.make_async_copy(ones, acc_d.at[didx.at[j]],
                                          sem_d).wait()

                plsc.subcore_barrier()
                pltpu.sync_copy(acc_s.at[pl.ds(row0, ROWS_SUB)],
                                cs_hbm.at[r, pl.ds(row0, ROWS_SUB)])
                pltpu.sync_copy(acc_d.at[pl.ds(row0, ROWS_SUB)],
                                cd_hbm.at[r, pl.ds(row0, ROWS_SUB)])

                pltpu.sync_copy(didx, do_hbm.at[r, sid])

                @pl.loop(0, NCH_SUB)
                def _(j):
                    for c4 in range(CH // 16):
                        sl = pl.ds(c4 * 16, 16)
                        sidx[j, sl] = sidx[j, sl] + jnp.full(
                            (16,), r * NP, jnp.int32)

                pltpu.sync_copy(sidx, so_hbm.at[r, sid])

    return k(s_idx, d_idx)


def _sc_edge_pass(z2d, s_idx_off, d_idx):
    @functools.partial(
        pl.kernel,
        out_type=jax.ShapeDtypeStruct((R, NP, H), jnp.float32),
        mesh=_get_mesh(),
        compiler_params=pltpu.CompilerParams(use_tc_tiling_on_sc=False),
        scratch_types=[
            pltpu.VMEM((NCH_SUB, CH), jnp.int32),
            pltpu.VMEM((NCH_SUB, CH), jnp.int32),
            pltpu.VMEM((NBUF, CH, H), jnp.float32),
            pltpu.VMEM((CH, H), jnp.float32),
            pltpu.VMEM_SHARED((NP, H), jnp.float32),
            [pltpu.SemaphoreType.DMA] * NBUF,
        ],
    )
    def k(z_hbm, s_hbm, d_hbm, agg_hbm, sidx, didx, rows, zb, acc, gsem):
        cid = lax.axis_index("c")
        sid = lax.axis_index("s")
        row0 = sid * ROWS_SUB

        @pl.loop(0, CH)
        def _(i):
            for c4 in range(H // 16):
                zb[i, pl.ds(c4 * 16, 16)] = jnp.zeros((16,), jnp.float32)

        for r in range(R):
            @pl.when(cid == OWNER[r])
            def _():
                for kz in range(ROWS_SUB // CH):
                    pltpu.sync_copy(zb, acc.at[pl.ds(row0 + kz * CH, CH)])
                pltpu.sync_copy(s_hbm.at[r, sid], sidx)
                pltpu.sync_copy(d_hbm.at[r, sid], didx)
                plsc.subcore_barrier()

                def wait_gather(c, b):
                    pltpu.make_async_copy(z_hbm.at[sidx.at[c]], rows.at[b],
                                          gsem[b]).wait()

                for b in range(NBUF):
                    pltpu.async_copy(z_hbm.at[sidx.at[b]], rows.at[b], gsem[b])

                @pl.loop(0, (NCH_SUB // NBUF) * NBUF, step=NBUF)
                def _(jj):
                    for b in range(NBUF):
                        c = jj + b
                        wait_gather(c, b)
                        pltpu.sync_copy(rows.at[b], acc.at[didx.at[c]],
                                        add=True)

                        @pl.when(c + NBUF < NCH_SUB)
                        def _():
                            pltpu.async_copy(z_hbm.at[sidx.at[c + NBUF]],
                                             rows.at[b], gsem[b])

                for c in range((NCH_SUB // NBUF) * NBUF, NCH_SUB):
                    b = c % NBUF
                    wait_gather(c, b)
                    pltpu.sync_copy(rows.at[b], acc.at[didx.at[c]], add=True)

                plsc.subcore_barrier()
                pltpu.sync_copy(acc.at[pl.ds(row0, ROWS_SUB)],
                                agg_hbm.at[r, pl.ds(row0, ROWS_SUB)])

    return k(z2d, s_idx_off, d_idx)


def _tc_z1(x0, x1, x2, x3, w1, cnt):
    def body(x0_ref, x1_ref, x2_ref, x3_ref, w_ref, c_ref, o_ref):
        xs = [x0_ref[...], x1_ref[...], x2_ref[...], x3_ref[...]]
        scl = lax.rsqrt(jnp.clip(c_ref[:, 0, :], 1.0, None))
        for r in range(R):
            y = jnp.dot(xs[SRC_OF[r]], w_ref[r],
                        preferred_element_type=jnp.float32)
            o_ref[r] = y * scl[r][:, None]

    x_bs = pl.BlockSpec((BLK, 128), lambda i: (i, 0))
    return pl.pallas_call(
        body,
        grid=(NBLK,),
        in_specs=[
            x_bs, x_bs, x_bs, x_bs,
            pl.BlockSpec((R, 128, H), lambda i: (0, 0, 0)),
            pl.BlockSpec((R, 1, BLK), lambda i: (0, 0, i)),
        ],
        out_specs=pl.BlockSpec((R, BLK, H), lambda i: (0, i, 0)),
        out_shape=jax.ShapeDtypeStruct((R, NP, H), jnp.float32),
    )(x0, x1, x2, x3, w1, cnt)


def _combine(agg, cnt_d, b):
    scl = lax.rsqrt(jnp.clip(cnt_d[:, 0, :], 1.0, None))
    conv = agg * scl[:, :, None] + b[:, None, :]
    rl = jax.nn.relu
    z = jnp.zeros((conv.shape[1], H), jnp.float32)
    h_cls = jnp.concatenate([rl(conv[0]), z, rl(conv[1])], axis=1)
    h_att = jnp.concatenate(
        [z, rl(jnp.maximum(jnp.maximum(conv[2], conv[3]), conv[4])), rl(conv[5])],
        axis=1)
    h_typ = jnp.concatenate(
        [z, rl(jnp.maximum(jnp.maximum(conv[6], conv[7]), conv[8])), rl(conv[9])],
        axis=1)
    h_val = jnp.concatenate([z, z, rl(conv[10])], axis=1)
    return [h_cls, h_att, h_typ, h_val]


def _tc_mid(agg1, cnt_d, cnt_s, b1, w2):
    def body(a_ref, cd_ref, cs_ref, b_ref, w_ref, o_ref):
        hs = _combine(a_ref[...], cd_ref[...], b_ref[...])
        scl_s = lax.rsqrt(jnp.clip(cs_ref[:, 0, :], 1.0, None))
        for r in range(R):
            y = jnp.dot(hs[SRC_OF[r]], w_ref[r],
                        preferred_element_type=jnp.float32)
            o_ref[r] = y * scl_s[r][:, None]

    return pl.pallas_call(
        body,
        grid=(NBLK,),
        in_specs=[
            pl.BlockSpec((R, BLK, H), lambda i: (0, i, 0)),
            pl.BlockSpec((R, 1, BLK), lambda i: (0, 0, i)),
            pl.BlockSpec((R, 1, BLK), lambda i: (0, 0, i)),
            pl.BlockSpec((R, H), lambda i: (0, 0)),
            pl.BlockSpec((R, 3 * H, H), lambda i: (0, 0, 0)),
        ],
        out_specs=pl.BlockSpec((R, BLK, H), lambda i: (0, i, 0)),
        out_shape=jax.ShapeDtypeStruct((R, NP, H), jnp.float32),
    )(agg1, cnt_d, cnt_s, b1, w2)


def _tc_final(agg2, cnt_d, b2):
    def body(a_ref, cd_ref, b_ref, oc_ref, oa_ref, ot_ref, ov_ref):
        hs = _combine(a_ref[...], cd_ref[...], b_ref[...])
        oc_ref[...], oa_ref[...], ot_ref[...], ov_ref[...] = hs

    out_bs = pl.BlockSpec((BLK, 3 * H), lambda i: (i, 0))
    out_t = jax.ShapeDtypeStruct((NP, 3 * H), jnp.float32)
    return pl.pallas_call(
        body,
        grid=(NBLK,),
        in_specs=[
            pl.BlockSpec((R, BLK, H), lambda i: (0, i, 0)),
            pl.BlockSpec((R, 1, BLK), lambda i: (0, 0, i)),
            pl.BlockSpec((R, H), lambda i: (0, 0)),
        ],
        out_specs=(out_bs, out_bs, out_bs, out_bs),
        out_shape=(out_t, out_t, out_t, out_t),
    )(agg2, cnt_d, b2)


def kernel(x_class, x_attribute, x_type, x_value,
           edges_a2c, edges_v2c, edges_c2a, edges_t2a, edges_v2a, edges_a2a,
           edges_c2t, edges_a2t, edges_v2t, edges_t2t, edges_c2v,
           W1, b1, W2, b2):
    edges = [edges_a2c, edges_v2c, edges_c2a, edges_t2a, edges_v2a, edges_a2a,
             edges_c2t, edges_a2t, edges_v2t, edges_t2t, edges_c2v]

    padvals = N + jnp.arange(EP - E, dtype=jnp.int32) % (NP - N)
    pad = jnp.stack([padvals, padvals])
    ee = jnp.stack([jnp.concatenate([e.astype(jnp.int32), pad], axis=1)
                    for e in edges])
    s_idx = ee[:, 0, :].reshape(R, NSUB, NCH_SUB, CH)
    d_idx = ee[:, 1, :].reshape(R, NSUB, NCH_SUB, CH)

    xs = [x_class, x_attribute, x_type, x_value]
    xs = [jnp.pad(x[:N], ((0, NP - N), (0, 0))) for x in xs]

    cs, cd, s_idx_off, d_idx = _sc_degrees(s_idx, d_idx)
    cnt_s = cs[:, :, 0].reshape(R, 1, NP)
    cnt_d = cd[:, :, 0].reshape(R, 1, NP)

    z1 = _tc_z1(xs[0], xs[1], xs[2], xs[3], W1, cnt_s).reshape(R * NP, H)

    agg1 = _sc_edge_pass(z1, s_idx_off, d_idx)

    z2 = _tc_mid(agg1, cnt_d, cnt_s, b1, W2).reshape(R * NP, H)

    agg2 = _sc_edge_pass(z2, s_idx_off, d_idx)

    oc, oa, ot, ov = _tc_final(agg2, cnt_d, b2)

    za = jnp.zeros((H,), jnp.float32)
    att_const = jax.nn.relu(jnp.concatenate(
        [za, jnp.maximum(jnp.maximum(b2[2], b2[3]), b2[4]), b2[5]]))
    attr_tail = jnp.broadcast_to(att_const, (N, 3 * H))

    return (oc[:N], jnp.concatenate([oa[:N], attr_tail], axis=0),
            ot[:N], ov[:N])
